# baseline (speedup 1.0000x reference)
import functools

import jax
import jax.numpy as jnp
from jax.experimental import pallas as pl
from jax.experimental.pallas import tpu as pltpu
from jax.experimental.pallas import tpu_sc as plsc

NA = 2048
NR = 2048
NDIM = 3
ATOM_NF = 16
RES_NF = 21
JOINT = 16
HID = 64
NL = 4
NBATCH = 32
NORM = 100.0
N = NA + NR
TILE = 128
NT = N // TILE
RT = 32
CT = 40


_SC_NW = 32
_BPW = NA // _SC_NW


def _sc_pick_rows(table, idx):
    mesh = plsc.VectorSubcoreMesh(core_axis_name="c", subcore_axis_name="s")

    @functools.partial(
        pl.kernel, mesh=mesh,
        out_type=jax.ShapeDtypeStruct((NA, 128), jnp.float32),
        scratch_types=[
            pltpu.VMEM((_BPW,), jnp.int32),
            pltpu.VMEM((_BPW, 128), jnp.float32),
            pltpu.SemaphoreType.DMA,
        ],
    )
    def k(table_hbm, idx_hbm, out_hbm, idx_v, rows_v, sem):
        wid = jax.lax.axis_index("s") * 2 + jax.lax.axis_index("c")
        base = wid * _BPW
        pltpu.sync_copy(idx_hbm.at[pl.ds(base, _BPW)], idx_v)
        pltpu.async_copy(table_hbm.at[idx_v], rows_v, sem).wait()
        pltpu.sync_copy(rows_v, out_hbm.at[pl.ds(base, _BPW)])

    return k(table, idx)


def _silu(x):
    return x * jax.nn.sigmoid(x)


def _egnn_kernel(ranges_ref, xa_ref, xr_ref, msort_ref, sperm_ref,
                 aew1, aeb1, aew2, aeb2, rew1, reb1, rew2, reb2,
                 embw, emb_bias, lw, outw, outb,
                 adw1, adb1, adw2, adb2, adw3, adb3,
                 out_ref, hh0_scr, hh_scr, a_scr, b_scr, agg_scr):
    ha = _silu(xa_ref[:, NDIM:] @ aew1[:] + aeb1[:]) @ aew2[:] + aeb2[:]
    hr = _silu(xr_ref[:, NDIM:] @ rew1[:] + reb1[:]) @ rew2[:] + reb2[:]
    hh0_scr[0:NA, :] = (xa_ref[:, :NDIM] @ embw[0:NDIM, :]
                        + ha @ embw[NDIM:NDIM + JOINT, :] + emb_bias[:])
    hh0_scr[NA:N, :] = (xr_ref[:, :NDIM] @ embw[0:NDIM, :]
                        + hr @ embw[NDIM:NDIM + JOINT, :] + emb_bias[:])

    def perm_body(r, _):
        pv = sperm_ref[pl.ds(r * TILE, TILE), :]
        colid = jax.lax.broadcasted_iota(jnp.int32, (TILE, N), 1)
        P = (pv == colid).astype(jnp.float32)
        hh_scr[pl.ds(r * TILE, TILE), :] = jax.lax.dot_general(
            P, hh0_scr[:], (((1,), (0,)), ((), ())),
            preferred_element_type=jnp.float32)
        return 0

    jax.lax.fori_loop(0, NT, perm_body, 0)

    for l in range(NL):
        w1, b1, w2, b2, nw1, nb1, nw2, nb2 = [r[:] for r in lw[l]]
        hh = hh_scr[:]
        a_scr[:] = hh @ w1[:HID, :] + b1
        b_scr[0:N, :] = hh @ w1[HID:, :]
        b_scr[N:N + CT, :] = jnp.zeros((CT, HID), jnp.float32)

        def row_body(r, _, w2=w2, b2=b2):
            a_tile = a_scr[pl.ds(r * RT, RT), :]
            mrow = msort_ref[pl.ds(r * RT, RT), :]
            base = ranges_ref[r, 0]

            def j_body(j, acc, base=base):
                b_tile = b_scr[pl.ds(base + j * CT, CT), :]
                mcol = msort_ref[pl.ds(base + j * CT, CT), :]
                s = _silu(a_tile[:, None, :] + b_tile[None, :, :])
                m = _silu(jax.lax.dot_general(
                    s, w2, (((2,), (0,)), ((), ())),
                    preferred_element_type=jnp.float32) + b2)
                adj = mrow[:, None, :] == mcol[None, :, :]
                return acc + jnp.sum(jnp.where(adj, m, 0.0), axis=1)

            acc = jnp.zeros((RT, HID), jnp.float32)
            acc = jax.lax.fori_loop(0, ranges_ref[r, 1], j_body, acc)
            agg_scr[pl.ds(r * RT, RT), :] = acc
            return 0

        jax.lax.fori_loop(0, N // RT, row_body, 0)

        hh = hh_scr[:]
        upd = _silu(hh @ nw1[:HID, :] + (agg_scr[:] * (1.0 / NORM)) @ nw1[HID:, :]
                    + nb1) @ nw2 + nb2
        hh_scr[:] = hh + upd

    out = hh_scr[:] @ outw[:] + outb[:]
    vel = out[:, :NDIM]
    hf = out[:, NDIM:NDIM + JOINT]
    seg_ids = jax.lax.broadcasted_iota(jnp.int32, (N, NBATCH), 1)
    oh = (msort_ref[0:N, :] == seg_ids).astype(jnp.float32)
    seg = jax.lax.dot_general(oh, vel, (((0,), (0,)), ((), ())),
                              preferred_element_type=jnp.float32)
    cnt = jnp.sum(oh, axis=0, keepdims=True)
    mean = seg / jnp.maximum(cnt.T, 1.0)
    velc = vel - oh @ mean
    d = _silu(hf @ adw1[:] + adb1[:])
    d = _silu(d @ adw2[:] + adb2[:])
    d = d @ adw3[:] + adb3[:]
    final = velc + d
    out_ref[:] = jnp.concatenate(
        [final, jnp.zeros((N, 125), jnp.float32)], axis=1)


def kernel(xh_atoms, xh_residues, t, mask_atoms, mask_residues, params):
    p = params
    ma = mask_atoms.astype(jnp.int32)
    mr = mask_residues.astype(jnp.int32)
    m_full = jnp.concatenate([ma, mr])

    perm = jnp.argsort(m_full, stable=True).astype(jnp.int32)
    inv = jnp.argsort(perm).astype(jnp.int32)
    msort = jnp.sort(m_full)
    apos = inv[:NA]

    mt = msort.reshape(N // RT, RT)
    lo = mt[:, 0]
    hi = mt[:, -1]
    s8 = ((jnp.searchsorted(msort, lo, side='left') // 8) * 8).astype(jnp.int32)
    e = jnp.searchsorted(msort, hi, side='right').astype(jnp.int32)
    nchunk = jnp.maximum(-((-(e - s8)) // CT), 0).astype(jnp.int32)
    ranges = jnp.stack([s8, nchunk], axis=1)

    emb_bias = (p['embb'] + t[0] * p['embw'][NDIM + JOINT])[None, :]

    lw_vals = []
    for l in range(NL):
        lw_vals.append([p[n % l] for n in
                        ('ew1_%d', 'eb1_%d', 'ew2_%d', 'eb2_%d',
                         'nw1_%d', 'nb1_%d', 'nw2_%d', 'nb2_%d')])

    flat_params = ([p['aew1'], p['aeb1'], p['aew2'], p['aeb2'],
                    p['rew1'], p['reb1'], p['rew2'], p['reb2'],
                    p['embw'], emb_bias]
                   + [w for layer in lw_vals for w in layer]
                   + [p['outw'], p['outb'],
                      p['adw1'], p['adb1'], p['adw2'], p['adb2'],
                      p['adw3'], p['adb3']])

    def kern_wrap(ranges_ref, xa_ref, xr_ref, msort_ref, sperm_ref, *rest):
        prm = list(rest[:len(flat_params)])
        out_ref = rest[len(flat_params)]
        scr = rest[len(flat_params) + 1:]
        aew1, aeb1, aew2, aeb2, rew1, reb1, rew2, reb2, embw, emb_b = prm[:10]
        lw = [prm[10 + 8 * l:10 + 8 * (l + 1)] for l in range(NL)]
        outw, outb, adw1, adb1, adw2, adb2, adw3, adb3 = prm[10 + 8 * NL:]
        _egnn_kernel(ranges_ref, xa_ref, xr_ref, msort_ref, sperm_ref,
                     aew1, aeb1, aew2, aeb2, rew1, reb1, rew2, reb2,
                     embw, emb_b, lw, outw, outb,
                     adw1, adb1, adw2, adb2, adw3, adb3,
                     out_ref, *scr)

    n_in = 5 + len(flat_params)
    in_specs = ([pl.BlockSpec(memory_space=pltpu.SMEM)]
                + [pl.BlockSpec(memory_space=pltpu.VMEM)] * (n_in - 1))
    out = pl.pallas_call(
        kern_wrap,
        out_shape=jax.ShapeDtypeStruct((N, 128), jnp.float32),
        in_specs=in_specs,
        out_specs=pl.BlockSpec(memory_space=pltpu.VMEM),
        scratch_shapes=[
            pltpu.VMEM((N, HID), jnp.float32),
            pltpu.VMEM((N, HID), jnp.float32),
            pltpu.VMEM((N, HID), jnp.float32),
            pltpu.VMEM((N + CT, HID), jnp.float32),
            pltpu.VMEM((N, HID), jnp.float32),
        ],
    )(ranges, xh_atoms, xh_residues,
      jnp.concatenate([msort, jnp.full((CT,), -1, jnp.int32)])[:, None],
      perm[:, None], *flat_params)
    return _sc_pick_rows(out, apos)[:, :NDIM]


# speedup vs baseline: 1.0127x; 1.0127x over previous
import functools

import jax
import jax.numpy as jnp
from jax.experimental import pallas as pl
from jax.experimental.pallas import tpu as pltpu
from jax.experimental.pallas import tpu_sc as plsc

NA = 2048
NR = 2048
NDIM = 3
ATOM_NF = 16
RES_NF = 21
JOINT = 16
HID = 64
NL = 4
NBATCH = 32
NORM = 100.0
N = NA + NR
TILE = 128
NT = N // TILE
RT = 32
CT = 48


_SC_NW = 32
_BPW = NA // _SC_NW


def _sc_pick_rows(table, idx):
    mesh = plsc.VectorSubcoreMesh(core_axis_name="c", subcore_axis_name="s")

    @functools.partial(
        pl.kernel, mesh=mesh,
        out_type=jax.ShapeDtypeStruct((NA, 128), jnp.float32),
        scratch_types=[
            pltpu.VMEM((_BPW,), jnp.int32),
            pltpu.VMEM((_BPW, 128), jnp.float32),
            pltpu.SemaphoreType.DMA,
        ],
    )
    def k(table_hbm, idx_hbm, out_hbm, idx_v, rows_v, sem):
        wid = jax.lax.axis_index("s") * 2 + jax.lax.axis_index("c")
        base = wid * _BPW
        pltpu.sync_copy(idx_hbm.at[pl.ds(base, _BPW)], idx_v)
        pltpu.async_copy(table_hbm.at[idx_v], rows_v, sem).wait()
        pltpu.sync_copy(rows_v, out_hbm.at[pl.ds(base, _BPW)])

    return k(table, idx)


def _silu(x):
    return x * jax.nn.sigmoid(x)


def _egnn_kernel(ranges_ref, xa_ref, xr_ref, msort_ref, sperm_ref,
                 aew1, aeb1, aew2, aeb2, rew1, reb1, rew2, reb2,
                 embw, emb_bias, lw, outw, outb,
                 adw1, adb1, adw2, adb2, adw3, adb3,
                 out_ref, hh0_scr, hh_scr, a_scr, b_scr, agg_scr):
    ha = _silu(xa_ref[:, NDIM:] @ aew1[:] + aeb1[:]) @ aew2[:] + aeb2[:]
    hr = _silu(xr_ref[:, NDIM:] @ rew1[:] + reb1[:]) @ rew2[:] + reb2[:]
    hh0_scr[0:NA, :] = (xa_ref[:, :NDIM] @ embw[0:NDIM, :]
                        + ha @ embw[NDIM:NDIM + JOINT, :] + emb_bias[:])
    hh0_scr[NA:N, :] = (xr_ref[:, :NDIM] @ embw[0:NDIM, :]
                        + hr @ embw[NDIM:NDIM + JOINT, :] + emb_bias[:])

    def perm_body(r, _):
        pv = sperm_ref[pl.ds(r * TILE, TILE), :]
        colid = jax.lax.broadcasted_iota(jnp.int32, (TILE, N), 1)
        P = (pv == colid).astype(jnp.float32)
        hh_scr[pl.ds(r * TILE, TILE), :] = jax.lax.dot_general(
            P, hh0_scr[:], (((1,), (0,)), ((), ())),
            preferred_element_type=jnp.float32)
        return 0

    jax.lax.fori_loop(0, NT, perm_body, 0)

    for l in range(NL):
        w1, b1, w2, b2, nw1, nb1, nw2, nb2 = [r[:] for r in lw[l]]
        hh = hh_scr[:]
        a_scr[:] = hh @ w1[:HID, :] + b1
        b_scr[0:N, :] = hh @ w1[HID:, :]
        b_scr[N:N + CT, :] = jnp.zeros((CT, HID), jnp.float32)

        def row_body(r, _, w2=w2, b2=b2):
            a_tile = a_scr[pl.ds(r * RT, RT), :]
            mrow = msort_ref[pl.ds(r * RT, RT), :]
            base = ranges_ref[r, 0]

            def j_body(j, acc, base=base):
                b_tile = b_scr[pl.ds(base + j * CT, CT), :]
                mcol = msort_ref[pl.ds(base + j * CT, CT), :]
                s = _silu(a_tile[:, None, :] + b_tile[None, :, :])
                m = _silu(jax.lax.dot_general(
                    s, w2, (((2,), (0,)), ((), ())),
                    preferred_element_type=jnp.float32) + b2)
                adj = mrow[:, None, :] == mcol[None, :, :]
                return acc + jnp.sum(jnp.where(adj, m, 0.0), axis=1)

            acc = jnp.zeros((RT, HID), jnp.float32)
            acc = jax.lax.fori_loop(0, ranges_ref[r, 1], j_body, acc)
            agg_scr[pl.ds(r * RT, RT), :] = acc
            return 0

        jax.lax.fori_loop(0, N // RT, row_body, 0)

        hh = hh_scr[:]
        upd = _silu(hh @ nw1[:HID, :] + (agg_scr[:] * (1.0 / NORM)) @ nw1[HID:, :]
                    + nb1) @ nw2 + nb2
        hh_scr[:] = hh + upd

    out = hh_scr[:] @ outw[:] + outb[:]
    vel = out[:, :NDIM]
    hf = out[:, NDIM:NDIM + JOINT]
    seg_ids = jax.lax.broadcasted_iota(jnp.int32, (N, NBATCH), 1)
    oh = (msort_ref[0:N, :] == seg_ids).astype(jnp.float32)
    seg = jax.lax.dot_general(oh, vel, (((0,), (0,)), ((), ())),
                              preferred_element_type=jnp.float32)
    cnt = jnp.sum(oh, axis=0, keepdims=True)
    mean = seg / jnp.maximum(cnt.T, 1.0)
    velc = vel - oh @ mean
    d = _silu(hf @ adw1[:] + adb1[:])
    d = _silu(d @ adw2[:] + adb2[:])
    d = d @ adw3[:] + adb3[:]
    final = velc + d
    out_ref[:] = jnp.concatenate(
        [final, jnp.zeros((N, 125), jnp.float32)], axis=1)


def kernel(xh_atoms, xh_residues, t, mask_atoms, mask_residues, params):
    p = params
    ma = mask_atoms.astype(jnp.int32)
    mr = mask_residues.astype(jnp.int32)
    m_full = jnp.concatenate([ma, mr])

    perm = jnp.argsort(m_full, stable=True).astype(jnp.int32)
    inv = jnp.argsort(perm).astype(jnp.int32)
    msort = jnp.sort(m_full)
    apos = inv[:NA]

    mt = msort.reshape(N // RT, RT)
    lo = mt[:, 0]
    hi = mt[:, -1]
    s8 = ((jnp.searchsorted(msort, lo, side='left') // 8) * 8).astype(jnp.int32)
    e = jnp.searchsorted(msort, hi, side='right').astype(jnp.int32)
    nchunk = jnp.maximum(-((-(e - s8)) // CT), 0).astype(jnp.int32)
    ranges = jnp.stack([s8, nchunk], axis=1)

    emb_bias = (p['embb'] + t[0] * p['embw'][NDIM + JOINT])[None, :]

    lw_vals = []
    for l in range(NL):
        lw_vals.append([p[n % l] for n in
                        ('ew1_%d', 'eb1_%d', 'ew2_%d', 'eb2_%d',
                         'nw1_%d', 'nb1_%d', 'nw2_%d', 'nb2_%d')])

    flat_params = ([p['aew1'], p['aeb1'], p['aew2'], p['aeb2'],
                    p['rew1'], p['reb1'], p['rew2'], p['reb2'],
                    p['embw'], emb_bias]
                   + [w for layer in lw_vals for w in layer]
                   + [p['outw'], p['outb'],
                      p['adw1'], p['adb1'], p['adw2'], p['adb2'],
                      p['adw3'], p['adb3']])

    def kern_wrap(ranges_ref, xa_ref, xr_ref, msort_ref, sperm_ref, *rest):
        prm = list(rest[:len(flat_params)])
        out_ref = rest[len(flat_params)]
        scr = rest[len(flat_params) + 1:]
        aew1, aeb1, aew2, aeb2, rew1, reb1, rew2, reb2, embw, emb_b = prm[:10]
        lw = [prm[10 + 8 * l:10 + 8 * (l + 1)] for l in range(NL)]
        outw, outb, adw1, adb1, adw2, adb2, adw3, adb3 = prm[10 + 8 * NL:]
        _egnn_kernel(ranges_ref, xa_ref, xr_ref, msort_ref, sperm_ref,
                     aew1, aeb1, aew2, aeb2, rew1, reb1, rew2, reb2,
                     embw, emb_b, lw, outw, outb,
                     adw1, adb1, adw2, adb2, adw3, adb3,
                     out_ref, *scr)

    n_in = 5 + len(flat_params)
    in_specs = ([pl.BlockSpec(memory_space=pltpu.SMEM)]
                + [pl.BlockSpec(memory_space=pltpu.VMEM)] * (n_in - 1))
    out = pl.pallas_call(
        kern_wrap,
        out_shape=jax.ShapeDtypeStruct((N, 128), jnp.float32),
        in_specs=in_specs,
        out_specs=pl.BlockSpec(memory_space=pltpu.VMEM),
        scratch_shapes=[
            pltpu.VMEM((N, HID), jnp.float32),
            pltpu.VMEM((N, HID), jnp.float32),
            pltpu.VMEM((N, HID), jnp.float32),
            pltpu.VMEM((N + CT, HID), jnp.float32),
            pltpu.VMEM((N, HID), jnp.float32),
        ],
    )(ranges, xh_atoms, xh_residues,
      jnp.concatenate([msort, jnp.full((CT,), -1, jnp.int32)])[:, None],
      perm[:, None], *flat_params)
    return _sc_pick_rows(out, apos)[:, :NDIM]
